# baseline (device time: 29986 ns/iter reference)
import jax
import jax.numpy as jnp
from jax import lax
from jax.experimental import pallas as pl
from jax.experimental.pallas import tpu as pltpu

N_DEV = 4


def kernel(x, w_mat):
    m, k_per = x.shape
    _, n = w_mat.shape
    m_out = m // N_DEV

    def body(x_ref, w_ref, out_ref, partial_ref, send_ref, recv_ref,
             send_sems, recv_sems):
        my = lax.axis_index("i")
        left = lax.rem(my + (N_DEV - 1), N_DEV)
        right = lax.rem(my + 1, N_DEV)

        barrier_sem = pltpu.get_barrier_semaphore()
        for nbr in (left, right):
            pl.semaphore_signal(
                barrier_sem, inc=1,
                device_id=(nbr,), device_id_type=pl.DeviceIdType.MESH,
            )
        pl.semaphore_wait(barrier_sem, 2)

        partial_ref[...] = jnp.dot(
            x_ref[...].astype(jnp.bfloat16),
            w_ref[...].astype(jnp.bfloat16),
            preferred_element_type=jnp.float32,
        )

        def chunk(c):
            return partial_ref[pl.ds(c * m_out, m_out), :]

        for t in range(N_DEV - 1):
            c_send = lax.rem(my + (N_DEV - 1 - t), N_DEV)
            if t == 0:
                send_ref[t, :, :] = chunk(c_send).astype(jnp.bfloat16)
            else:
                send_ref[t, :, :] = (
                    recv_ref[t - 1].astype(jnp.float32) + chunk(c_send)
                ).astype(jnp.bfloat16)
            rdma = pltpu.make_async_remote_copy(
                src_ref=send_ref.at[t],
                dst_ref=recv_ref.at[t],
                send_sem=send_sems.at[t],
                recv_sem=recv_sems.at[t],
                device_id=(right,),
                device_id_type=pl.DeviceIdType.MESH,
            )
            rdma.start()
            rdma.wait()

        y = recv_ref[N_DEV - 2].astype(jnp.float32) + chunk(my)
        out_ref[...] = y * jax.nn.sigmoid(y)

    return pl.pallas_call(
        body,
        out_shape=jax.ShapeDtypeStruct((m_out, n), jnp.float32),
        in_specs=[
            pl.BlockSpec(memory_space=pltpu.VMEM),
            pl.BlockSpec(memory_space=pltpu.VMEM),
        ],
        out_specs=pl.BlockSpec(memory_space=pltpu.VMEM),
        scratch_shapes=[
            pltpu.VMEM((m, n), jnp.float32),
            pltpu.VMEM((N_DEV - 1, m_out, n), jnp.bfloat16),
            pltpu.VMEM((N_DEV - 1, m_out, n), jnp.bfloat16),
            pltpu.SemaphoreType.DMA((N_DEV - 1,)),
            pltpu.SemaphoreType.DMA((N_DEV - 1,)),
        ],
        compiler_params=pltpu.CompilerParams(collective_id=0),
    )(x, w_mat)


# device time: 17727 ns/iter; 1.6915x vs baseline; 1.6915x over previous
import jax
import jax.numpy as jnp
from jax import lax
from jax.experimental import pallas as pl
from jax.experimental.pallas import tpu as pltpu

N_DEV = 4
SEGS = 2


def kernel(x, w_mat):
    m, k_per = x.shape
    _, n = w_mat.shape
    m_out = m // N_DEV
    half = n // 2
    seg = half // SEGS

    def body(x_ref, w_ref, out_ref, partial_ref, send_ref, recv_ref,
             send_sems, recv_sems):
        my = lax.axis_index("i")
        left = lax.rem(my + (N_DEV - 1), N_DEV)
        right = lax.rem(my + 1, N_DEV)

        barrier_sem = pltpu.get_barrier_semaphore()
        for nbr in (left, right):
            pl.semaphore_signal(
                barrier_sem, inc=1,
                device_id=(nbr,), device_id_type=pl.DeviceIdType.MESH,
            )

        partial_ref[...] = jnp.dot(
            x_ref[...].astype(jnp.bfloat16),
            w_ref[...].astype(jnp.bfloat16),
            preferred_element_type=jnp.float32,
        )

        pl.semaphore_wait(barrier_sem, 2)

        def rows(d, t):
            c = lax.rem(my + (N_DEV - 1 - t), N_DEV) if d == 0 else \
                lax.rem(my + 1 + t, N_DEV)
            return pl.ds(c * m_out, m_out)

        def pcols(d, s):
            return pl.ds(d * half + s * seg, seg)

        def bcols(s):
            return pl.ds(s * seg, seg)

        def make_rdma(d, t, s):
            return pltpu.make_async_remote_copy(
                src_ref=send_ref.at[d, t, :, bcols(s)],
                dst_ref=recv_ref.at[d, t, :, bcols(s)],
                send_sem=send_sems.at[d, t, s],
                recv_sem=recv_sems.at[d, t, s],
                device_id=(right if d == 0 else left,),
                device_id_type=pl.DeviceIdType.MESH,
            )

        rdmas = {}

        for s in range(SEGS):
            for d in range(2):
                send_ref[d, 0, :, bcols(s)] = (
                    partial_ref[rows(d, 0), pcols(d, s)].astype(jnp.bfloat16)
                )
                r = rdmas[(d, 0, s)] = make_rdma(d, 0, s)
                r.start()

        for t in range(1, N_DEV - 1):
            for s in range(SEGS):
                for d in range(2):
                    rdmas[(d, t - 1, s)].wait_recv()
                    acc = (
                        recv_ref[d, t - 1, :, bcols(s)].astype(jnp.float32)
                        + partial_ref[rows(d, t), pcols(d, s)]
                    )
                    send_ref[d, t, :, bcols(s)] = acc.astype(jnp.bfloat16)
                    r = rdmas[(d, t, s)] = make_rdma(d, t, s)
                    r.start()

        for s in range(SEGS):
            for d in range(2):
                rdmas[(d, N_DEV - 2, s)].wait_recv()
                y = (
                    recv_ref[d, N_DEV - 2, :, bcols(s)].astype(jnp.float32)
                    + partial_ref[pl.ds(my * m_out, m_out), pcols(d, s)]
                )
                out_ref[:, pcols(d, s)] = y * jax.nn.sigmoid(y)

        for r in rdmas.values():
            r.wait_send()

    return pl.pallas_call(
        body,
        out_shape=jax.ShapeDtypeStruct((m_out, n), jnp.float32),
        in_specs=[
            pl.BlockSpec(memory_space=pltpu.VMEM),
            pl.BlockSpec(memory_space=pltpu.VMEM),
        ],
        out_specs=pl.BlockSpec(memory_space=pltpu.VMEM),
        scratch_shapes=[
            pltpu.VMEM((m, n), jnp.float32),
            pltpu.VMEM((2, N_DEV - 1, m_out, half), jnp.bfloat16),
            pltpu.VMEM((2, N_DEV - 1, m_out, half), jnp.bfloat16),
            pltpu.SemaphoreType.DMA((2, N_DEV - 1, SEGS)),
            pltpu.SemaphoreType.DMA((2, N_DEV - 1, SEGS)),
        ],
        compiler_params=pltpu.CompilerParams(collective_id=0),
    )(x, w_mat)
